# baseline (speedup 1.0000x reference)
import functools

import jax
import jax.numpy as jnp
from jax import lax
from jax.experimental import pallas as pl
from jax.experimental.pallas import tpu as pltpu
from jax.experimental.pallas import tpu_sc as plsc

NC = 1
NS = 16
NW = NC * NS
K = 128


def _mm1_body(h_ref, w1_ref, wv_ref, x1_ref, xs_ref):
    x1 = jnp.dot(h_ref[...], w1_ref[...], preferred_element_type=jnp.float32)
    x1_ref[...] = x1
    xs_ref[...] = x1 * wv_ref[...]


def _mid_body(agg_ref, x1_ref, b1_ref, wv_ref, h1_ref, hs1_ref):
    a = agg_ref[0]
    for c in range(1, NC):
        a = a + agg_ref[c]
    s = a + x1_ref[...] + b1_ref[...]
    h1 = jnp.maximum(s, 0.0)
    h1_ref[...] = h1
    hs1_ref[...] = h1 * wv_ref[...]


def _out_body(agg_ref, h1_ref, w2_ref, b2_ref, o_ref):
    z = agg_ref[0]
    for c in range(1, NC):
        z = z + agg_ref[c]
    z = z + h1_ref[...]
    z = jnp.dot(z, w2_ref[...], preferred_element_type=jnp.float32) + b2_ref[...]
    z = z - jnp.max(z, axis=1, keepdims=True)
    o_ref[...] = z - jnp.log(jnp.sum(jnp.exp(z), axis=1, keepdims=True))


NBUF = 8


def _mp_body(nec, npt, hid, table, srcr, dstr, out, sidx, didx, rows, acc,
             *sems):
    semg, sems_ = sems[:NBUF], sems[NBUF:]
    cid = lax.axis_index("c")
    sid = lax.axis_index("s")
    pltpu.sync_copy(srcr.at[cid, sid], sidx)
    pltpu.sync_copy(dstr.at[cid, sid], didx)
    zero = jnp.zeros((hid,), jnp.float32)
    for i in range(K):
        rows[0, i, :] = zero
    for k in range(npt // K):
        pltpu.sync_copy(rows.at[0], acc.at[pl.ds(sid * npt + k * K, K)])
    plsc.subcore_barrier()

    for b in range(NBUF):
        pltpu.async_copy(table.at[sidx.at[b]], rows.at[b], semg[b])

    def outer(o, carry):
        base = o * NBUF
        for b in range(NBUF):
            j = base + b
            pltpu.make_async_copy(table.at[sidx.at[j]], rows.at[b],
                                  semg[b]).wait()
            pltpu.async_copy(rows.at[b], acc.at[didx.at[j]], sems_[b],
                             add=True)
        for b in range(NBUF):
            j = base + b
            pltpu.make_async_copy(rows.at[b], acc.at[didx.at[j]],
                                  sems_[b]).wait()
            jn = j + NBUF

            @pl.when(jn < nec)
            def _():
                pltpu.async_copy(table.at[sidx.at[jn]], rows.at[b], semg[b])
        return carry

    lax.fori_loop(0, nec // NBUF, outer, 0)
    plsc.subcore_barrier()
    pltpu.sync_copy(acc.at[pl.ds(sid * npt, npt)],
                    out.at[cid, pl.ds(sid * npt, npt)])


@jax.jit
def kernel(structure, H, input_weight, W1, b1, W2, b2):
    N, D = H.shape
    HID = W1.shape[1]
    C = W2.shape[1]
    E = structure.shape[1]

    NP = ((N + 1 + NS * K - 1) // (NS * K)) * (NS * K)
    EPW = -(-E // (NW * K * NBUF)) * (K * NBUF)
    NEC = EPW // K
    NPT = NP // NS

    wv = input_weight[:, None]
    edges = jnp.pad(structure, ((0, 0), (0, EPW * NW - E)),
                    constant_values=N)
    srcr = edges[0].reshape(NC, NS, NEC, K)
    dstr = edges[1].reshape(NC, NS, NEC, K)

    B = 2048
    grid = NP // B

    x1, xs = pl.pallas_call(
        _mm1_body,
        grid=(grid,),
        in_specs=[
            pl.BlockSpec((B, D), lambda i: (i, 0)),
            pl.BlockSpec((D, HID), lambda i: (0, 0)),
            pl.BlockSpec((B, 1), lambda i: (i, 0)),
        ],
        out_specs=[
            pl.BlockSpec((B, HID), lambda i: (i, 0)),
            pl.BlockSpec((B, HID), lambda i: (i, 0)),
        ],
        out_shape=[
            jax.ShapeDtypeStruct((NP, HID), jnp.float32),
            jax.ShapeDtypeStruct((NP, HID), jnp.float32),
        ],
    )(H, W1, wv)

    mp = pl.kernel(
        functools.partial(_mp_body, NEC, NPT, HID),
        out_type=jax.ShapeDtypeStruct((NC, NP, HID), jnp.float32),
        mesh=plsc.VectorSubcoreMesh(
            core_axis_name="c", subcore_axis_name="s",
            num_cores=NC, num_subcores=NS),
        scratch_types=[
            pltpu.VMEM((NEC, K), jnp.int32),
            pltpu.VMEM((NEC, K), jnp.int32),
            pltpu.VMEM((NBUF, K, HID), jnp.float32),
            pltpu.VMEM_SHARED((NP, HID), jnp.float32),
        ] + [pltpu.SemaphoreType.DMA] * (2 * NBUF),
        compiler_params=pltpu.CompilerParams(use_tc_tiling_on_sc=False),
    )

    agg1 = mp(xs, srcr, dstr)

    h1, hs1 = pl.pallas_call(
        _mid_body,
        grid=(grid,),
        in_specs=[
            pl.BlockSpec((NC, B, HID), lambda i: (0, i, 0)),
            pl.BlockSpec((B, HID), lambda i: (i, 0)),
            pl.BlockSpec((1, HID), lambda i: (0, 0)),
            pl.BlockSpec((B, 1), lambda i: (i, 0)),
        ],
        out_specs=[
            pl.BlockSpec((B, HID), lambda i: (i, 0)),
            pl.BlockSpec((B, HID), lambda i: (i, 0)),
        ],
        out_shape=[
            jax.ShapeDtypeStruct((NP, HID), jnp.float32),
            jax.ShapeDtypeStruct((NP, HID), jnp.float32),
        ],
    )(agg1, x1, b1[None, :], wv)

    agg2 = mp(hs1, srcr, dstr)

    out = pl.pallas_call(
        _out_body,
        grid=(grid,),
        in_specs=[
            pl.BlockSpec((NC, B, HID), lambda i: (0, i, 0)),
            pl.BlockSpec((B, HID), lambda i: (i, 0)),
            pl.BlockSpec((HID, C), lambda i: (0, 0)),
            pl.BlockSpec((1, C), lambda i: (0, 0)),
        ],
        out_specs=pl.BlockSpec((B, C), lambda i: (i, 0)),
        out_shape=jax.ShapeDtypeStruct((NP, C), jnp.float32),
    )(agg2, h1, W2, b2[None, :])

    return out[:N]


# speedup vs baseline: 1.2297x; 1.2297x over previous
import functools

import jax
import jax.numpy as jnp
from jax import lax
from jax.experimental import pallas as pl
from jax.experimental.pallas import tpu as pltpu
from jax.experimental.pallas import tpu_sc as plsc

NC = 1
NS = 16
NW = NC * NS
K = 128
NBUF = 8


def _mm1_body(h_ref, w1_ref, wv_ref, b1_ref, x1b_ref, xs_ref, wm_ref):
    x1 = jnp.dot(h_ref[...], w1_ref[...], preferred_element_type=jnp.float32)
    wv = wv_ref[...]
    x1b_ref[...] = x1 + b1_ref[...]
    xs_ref[...] = x1 * wv
    wm_ref[...] = jnp.broadcast_to(wv, x1.shape)


def _out_body(agg_ref, h1_ref, w2_ref, b2_ref, o_ref):
    z = agg_ref[...] + h1_ref[...]
    z = jnp.dot(z, w2_ref[...], preferred_element_type=jnp.float32) + b2_ref[...]
    z = z - jnp.max(z, axis=1, keepdims=True)
    o_ref[...] = z - jnp.log(jnp.sum(jnp.exp(z), axis=1, keepdims=True))


def _edge_loop(nec, table, sidx, didx, rows, acc, semg, sems_):
    for b in range(NBUF):
        pltpu.async_copy(table.at[sidx.at[b]], rows.at[b], semg[b])

    def outer(o, carry):
        base = o * NBUF
        for b in range(NBUF):
            j = base + b
            pltpu.make_async_copy(table.at[sidx.at[j]], rows.at[b],
                                  semg[b]).wait()
            pltpu.async_copy(rows.at[b], acc.at[didx.at[j]], sems_[b],
                             add=True)
        for b in range(NBUF):
            j = base + b
            pltpu.make_async_copy(rows.at[b], acc.at[didx.at[j]],
                                  sems_[b]).wait()
            jn = j + NBUF

            @pl.when(jn < nec)
            def _():
                pltpu.async_copy(table.at[sidx.at[jn]], rows.at[b], semg[b])
        return carry

    lax.fori_loop(0, nec // NBUF, outer, 0)


def _mega_body(nec, npt, hid, xs, x1b, wm, srcr, dstr, h1_out, agg2_out,
               sidx, didx, rows, zbuf, av, bv, cv, acc, tbl2, *sems):
    semg, sems_ = sems[:NBUF], sems[NBUF:]
    sid = lax.axis_index("s")
    r0 = sid * npt
    pltpu.sync_copy(srcr.at[0, sid], sidx)
    pltpu.sync_copy(dstr.at[0, sid], didx)
    zero = jnp.zeros((hid,), jnp.float32)
    for i in range(K):
        zbuf[i, :] = zero
    for k in range(npt // K):
        pltpu.sync_copy(zbuf, acc.at[pl.ds(r0 + k * K, K)])
    plsc.subcore_barrier()

    _edge_loop(nec, xs, sidx, didx, rows, acc, semg, sems_)
    plsc.subcore_barrier()

    pltpu.sync_copy(acc.at[pl.ds(r0, npt)], av)
    pltpu.sync_copy(x1b.at[pl.ds(r0, npt)], bv)
    pltpu.sync_copy(wm.at[pl.ds(r0, npt)], cv)

    def mrow(i, carry):
        h = jnp.maximum(av[i, :] + bv[i, :], 0.0)
        bv[i, :] = h
        cv[i, :] = h * cv[i, :]
        return carry

    lax.fori_loop(0, npt, mrow, 0)
    pltpu.sync_copy(bv, h1_out.at[pl.ds(r0, npt)])
    pltpu.sync_copy(cv, tbl2.at[pl.ds(r0, npt)])
    for k in range(npt // K):
        pltpu.sync_copy(zbuf, acc.at[pl.ds(r0 + k * K, K)])
    plsc.subcore_barrier()

    _edge_loop(nec, tbl2, sidx, didx, rows, acc, semg, sems_)
    plsc.subcore_barrier()
    pltpu.sync_copy(acc.at[pl.ds(r0, npt)], agg2_out.at[pl.ds(r0, npt)])


@jax.jit
def kernel(structure, H, input_weight, W1, b1, W2, b2):
    N, D = H.shape
    HID = W1.shape[1]
    C = W2.shape[1]
    E = structure.shape[1]

    NP = ((N + 1 + NS * K - 1) // (NS * K)) * (NS * K)
    EPW = -(-E // (NW * K * NBUF)) * (K * NBUF)
    NEC = EPW // K
    NPT = NP // NS

    wv = jnp.pad(input_weight, (0, NP - N))[:, None]
    edges = jnp.pad(structure, ((0, 0), (0, EPW * NW - E)),
                    constant_values=N)
    srcr = edges[0].reshape(NC, NS, NEC, K)
    dstr = edges[1].reshape(NC, NS, NEC, K)

    B = 2048
    grid = NP // B

    x1b, xs, wm = pl.pallas_call(
        _mm1_body,
        grid=(grid,),
        in_specs=[
            pl.BlockSpec((B, D), lambda i: (i, 0)),
            pl.BlockSpec((D, HID), lambda i: (0, 0)),
            pl.BlockSpec((B, 1), lambda i: (i, 0)),
            pl.BlockSpec((1, HID), lambda i: (0, 0)),
        ],
        out_specs=[
            pl.BlockSpec((B, HID), lambda i: (i, 0)),
            pl.BlockSpec((B, HID), lambda i: (i, 0)),
            pl.BlockSpec((B, HID), lambda i: (i, 0)),
        ],
        out_shape=[
            jax.ShapeDtypeStruct((NP, HID), jnp.float32),
            jax.ShapeDtypeStruct((NP, HID), jnp.float32),
            jax.ShapeDtypeStruct((NP, HID), jnp.float32),
        ],
    )(H, W1, wv, b1[None, :])

    h1, agg2 = pl.kernel(
        functools.partial(_mega_body, NEC, NPT, HID),
        out_type=[
            jax.ShapeDtypeStruct((NP, HID), jnp.float32),
            jax.ShapeDtypeStruct((NP, HID), jnp.float32),
        ],
        mesh=plsc.VectorSubcoreMesh(
            core_axis_name="c", subcore_axis_name="s",
            num_cores=NC, num_subcores=NS),
        scratch_types=[
            pltpu.VMEM((NEC, K), jnp.int32),
            pltpu.VMEM((NEC, K), jnp.int32),
            pltpu.VMEM((NBUF, K, HID), jnp.float32),
            pltpu.VMEM((K, HID), jnp.float32),
            pltpu.VMEM((NPT, HID), jnp.float32),
            pltpu.VMEM((NPT, HID), jnp.float32),
            pltpu.VMEM((NPT, HID), jnp.float32),
            pltpu.VMEM_SHARED((NP, HID), jnp.float32),
            pltpu.VMEM_SHARED((NP, HID), jnp.float32),
        ] + [pltpu.SemaphoreType.DMA] * (2 * NBUF),
        compiler_params=pltpu.CompilerParams(use_tc_tiling_on_sc=False),
    )(xs, x1b, wm, srcr, dstr)

    out = pl.pallas_call(
        _out_body,
        grid=(grid,),
        in_specs=[
            pl.BlockSpec((B, HID), lambda i: (i, 0)),
            pl.BlockSpec((B, HID), lambda i: (i, 0)),
            pl.BlockSpec((HID, C), lambda i: (0, 0)),
            pl.BlockSpec((1, C), lambda i: (0, 0)),
        ],
        out_specs=pl.BlockSpec((B, C), lambda i: (i, 0)),
        out_shape=jax.ShapeDtypeStruct((NP, C), jnp.float32),
    )(agg2, h1, W2, b2[None, :])

    return out[:N]


# speedup vs baseline: 1.4115x; 1.1478x over previous
import functools

import jax
import jax.numpy as jnp
from jax import lax
from jax.experimental import pallas as pl
from jax.experimental.pallas import tpu as pltpu
from jax.experimental.pallas import tpu_sc as plsc

NC = 1
NS = 16
NW = NC * NS
K = 128
NBUF = 8


def _mm1_body(h_ref, w1_ref, wv_ref, b1_ref, x1b_ref, xs_ref, wm_ref):
    x1 = jnp.dot(h_ref[...], w1_ref[...], preferred_element_type=jnp.float32)
    wv = wv_ref[...]
    x1b_ref[...] = x1 + b1_ref[...]
    xs_ref[...] = x1 * wv
    wm_ref[...] = jnp.broadcast_to(wv, x1.shape)


def _out_body(agg_ref, h1_ref, w2_ref, b2_ref, o_ref):
    z = agg_ref[...] + h1_ref[...]
    z = jnp.dot(z, w2_ref[...], preferred_element_type=jnp.float32) + b2_ref[...]
    z = z - jnp.max(z, axis=1, keepdims=True)
    o_ref[...] = z - jnp.log(jnp.sum(jnp.exp(z), axis=1, keepdims=True))


def _edge_loop(nec, table, sidx, didx, rows, acc, semg, sems_):
    for b in range(NBUF):
        pltpu.async_copy(table.at[sidx.at[b]], rows.at[b], semg[b])

    def outer(o, carry):
        base = o * NBUF
        for b in range(NBUF):
            j = base + b
            pltpu.make_async_copy(table.at[sidx.at[j]], rows.at[b],
                                  semg[b]).wait()
            pltpu.async_copy(rows.at[b], acc.at[didx.at[j]], sems_[b],
                             add=True)
        for b in range(NBUF):
            j = base + b
            pltpu.make_async_copy(rows.at[b], acc.at[didx.at[j]],
                                  sems_[b]).wait()
            jn = j + NBUF

            @pl.when(jn < nec)
            def _():
                pltpu.async_copy(table.at[sidx.at[jn]], rows.at[b], semg[b])
        return carry

    lax.fori_loop(0, nec // NBUF, outer, 0)


def _mega_body(nec, npt, hid, xs, x1b, wm, srcr, dstr, h1_out, agg2_out,
               sidx, didx, rows, zbuf, av, bv, cv, acc, tbl1, tbl2, *sems):
    semg, sems_ = sems[:NBUF], sems[NBUF:]
    sid = lax.axis_index("s")
    r0 = sid * npt
    pltpu.sync_copy(srcr.at[0, sid], sidx)
    pltpu.sync_copy(dstr.at[0, sid], didx)
    pltpu.sync_copy(xs.at[pl.ds(r0, npt)], tbl1.at[pl.ds(r0, npt)])
    zero = jnp.zeros((hid,), jnp.float32)
    for i in range(K):
        zbuf[i, :] = zero
    for k in range(npt // K):
        pltpu.sync_copy(zbuf, acc.at[pl.ds(r0 + k * K, K)])
    plsc.subcore_barrier()

    _edge_loop(nec, tbl1, sidx, didx, rows, acc, semg, sems_)
    plsc.subcore_barrier()

    pltpu.sync_copy(acc.at[pl.ds(r0, npt)], av)
    pltpu.sync_copy(x1b.at[pl.ds(r0, npt)], bv)
    pltpu.sync_copy(wm.at[pl.ds(r0, npt)], cv)

    def mrow(i, carry):
        h = jnp.maximum(av[i, :] + bv[i, :], 0.0)
        bv[i, :] = h
        cv[i, :] = h * cv[i, :]
        return carry

    lax.fori_loop(0, npt, mrow, 0)
    pltpu.sync_copy(bv, h1_out.at[pl.ds(r0, npt)])
    pltpu.sync_copy(cv, tbl2.at[pl.ds(r0, npt)])
    for k in range(npt // K):
        pltpu.sync_copy(zbuf, acc.at[pl.ds(r0 + k * K, K)])
    plsc.subcore_barrier()

    _edge_loop(nec, tbl2, sidx, didx, rows, acc, semg, sems_)
    plsc.subcore_barrier()
    pltpu.sync_copy(acc.at[pl.ds(r0, npt)], agg2_out.at[pl.ds(r0, npt)])


@jax.jit
def kernel(structure, H, input_weight, W1, b1, W2, b2):
    N, D = H.shape
    HID = W1.shape[1]
    C = W2.shape[1]
    E = structure.shape[1]

    NP = ((N + 1 + NS * K - 1) // (NS * K)) * (NS * K)
    EPW = -(-E // (NW * K * NBUF)) * (K * NBUF)
    NEC = EPW // K
    NPT = NP // NS

    wv = jnp.pad(input_weight, (0, NP - N))[:, None]
    edges = jnp.pad(structure, ((0, 0), (0, EPW * NW - E)),
                    constant_values=N)
    srcr = edges[0].reshape(NC, NS, NEC, K)
    dstr = edges[1].reshape(NC, NS, NEC, K)

    B = 2048
    grid = NP // B

    x1b, xs, wm = pl.pallas_call(
        _mm1_body,
        grid=(grid,),
        in_specs=[
            pl.BlockSpec((B, D), lambda i: (i, 0)),
            pl.BlockSpec((D, HID), lambda i: (0, 0)),
            pl.BlockSpec((B, 1), lambda i: (i, 0)),
            pl.BlockSpec((1, HID), lambda i: (0, 0)),
        ],
        out_specs=[
            pl.BlockSpec((B, HID), lambda i: (i, 0)),
            pl.BlockSpec((B, HID), lambda i: (i, 0)),
            pl.BlockSpec((B, HID), lambda i: (i, 0)),
        ],
        out_shape=[
            jax.ShapeDtypeStruct((NP, HID), jnp.float32),
            jax.ShapeDtypeStruct((NP, HID), jnp.float32),
            jax.ShapeDtypeStruct((NP, HID), jnp.float32),
        ],
    )(H, W1, wv, b1[None, :])

    h1, agg2 = pl.kernel(
        functools.partial(_mega_body, NEC, NPT, HID),
        out_type=[
            jax.ShapeDtypeStruct((NP, HID), jnp.float32),
            jax.ShapeDtypeStruct((NP, HID), jnp.float32),
        ],
        mesh=plsc.VectorSubcoreMesh(
            core_axis_name="c", subcore_axis_name="s",
            num_cores=NC, num_subcores=NS),
        scratch_types=[
            pltpu.VMEM((NEC, K), jnp.int32),
            pltpu.VMEM((NEC, K), jnp.int32),
            pltpu.VMEM((NBUF, K, HID), jnp.float32),
            pltpu.VMEM((K, HID), jnp.float32),
            pltpu.VMEM((NPT, HID), jnp.float32),
            pltpu.VMEM((NPT, HID), jnp.float32),
            pltpu.VMEM((NPT, HID), jnp.float32),
            pltpu.VMEM_SHARED((NP, HID), jnp.float32),
            pltpu.VMEM_SHARED((NP, HID), jnp.float32),
            pltpu.VMEM_SHARED((NP, HID), jnp.float32),
        ] + [pltpu.SemaphoreType.DMA] * (2 * NBUF),
        compiler_params=pltpu.CompilerParams(use_tc_tiling_on_sc=False),
    )(xs, x1b, wm, srcr, dstr)

    out = pl.pallas_call(
        _out_body,
        grid=(grid,),
        in_specs=[
            pl.BlockSpec((B, HID), lambda i: (i, 0)),
            pl.BlockSpec((B, HID), lambda i: (i, 0)),
            pl.BlockSpec((HID, C), lambda i: (0, 0)),
            pl.BlockSpec((1, C), lambda i: (0, 0)),
        ],
        out_specs=pl.BlockSpec((B, C), lambda i: (i, 0)),
        out_shape=jax.ShapeDtypeStruct((NP, C), jnp.float32),
    )(agg2, h1, W2, b2[None, :])

    return out[:N]


# speedup vs baseline: 1.4150x; 1.0025x over previous
import functools

import jax
import jax.numpy as jnp
from jax import lax
from jax.experimental import pallas as pl
from jax.experimental.pallas import tpu as pltpu
from jax.experimental.pallas import tpu_sc as plsc

NC = 1
NS = 16
NW = NC * NS
K = 128
NBUF = 8


def _mm1_body(h_ref, w1_ref, wv_ref, b1_ref, x1b_ref, xs_ref, wm_ref):
    x1 = jnp.dot(h_ref[...], w1_ref[...], preferred_element_type=jnp.float32)
    wv = wv_ref[...]
    x1b_ref[...] = x1 + b1_ref[...]
    xs_ref[...] = x1 * wv
    wm_ref[...] = jnp.broadcast_to(wv, x1.shape)


def _out_body(agg_ref, h1_ref, w2_ref, b2_ref, o_ref):
    z = agg_ref[...] + h1_ref[...]
    z = jnp.dot(z, w2_ref[...], preferred_element_type=jnp.float32) + b2_ref[...]
    z = z - jnp.max(z, axis=1, keepdims=True)
    o_ref[...] = z - jnp.log(jnp.sum(jnp.exp(z), axis=1, keepdims=True))


def _edge_loop(nec, table, sidx, didx, rows, acc, semg, sems_):
    for b in range(NBUF):
        pltpu.async_copy(table.at[sidx.at[b]], rows.at[b], semg[b])

    def outer(o, carry):
        base = o * NBUF
        for b in range(NBUF):
            j = base + b
            pltpu.make_async_copy(table.at[sidx.at[j]], rows.at[b],
                                  semg[b]).wait()
            pltpu.async_copy(rows.at[b], acc.at[didx.at[j]], sems_[b],
                             add=True)
        for b in range(NBUF):
            j = base + b
            pltpu.make_async_copy(rows.at[b], acc.at[didx.at[j]],
                                  sems_[b]).wait()
            jn = j + NBUF

            @pl.when(jn < nec)
            def _():
                pltpu.async_copy(table.at[sidx.at[jn]], rows.at[b], semg[b])
        return carry

    lax.fori_loop(0, nec // NBUF, outer, 0)


def _mega_body(nec, npt, hid, xs, x1b, wm, srcr, dstr, h1_out, agg2_out,
               sidx, didx, rows, zbuf, av, bv, cv, acc, tbl1, tbl2, *sems):
    semg, sems_ = sems[:NBUF], sems[NBUF:]
    sid = lax.axis_index("s")
    r0 = sid * npt
    pltpu.sync_copy(srcr.at[0, sid], sidx)
    pltpu.sync_copy(dstr.at[0, sid], didx)
    pltpu.sync_copy(xs.at[pl.ds(r0, npt)], tbl1.at[pl.ds(r0, npt)])
    zero = jnp.zeros((hid,), jnp.float32)
    for i in range(K):
        zbuf[i, :] = zero
    for k in range(npt // K):
        pltpu.sync_copy(zbuf, acc.at[pl.ds(r0 + k * K, K)])
    plsc.subcore_barrier()

    _edge_loop(nec, tbl1, sidx, didx, rows, acc, semg, sems_)
    plsc.subcore_barrier()

    pltpu.sync_copy(acc.at[pl.ds(r0, npt)], av)
    pltpu.sync_copy(x1b.at[pl.ds(r0, npt)], bv)
    pltpu.sync_copy(wm.at[pl.ds(r0, npt)], cv)

    def mrow(i, carry):
        h = jnp.maximum(av[i, :] + bv[i, :], 0.0)
        bv[i, :] = h
        cv[i, :] = h * cv[i, :]
        return carry

    lax.fori_loop(0, npt, mrow, 0)
    pltpu.sync_copy(bv, h1_out.at[pl.ds(r0, npt)])
    pltpu.sync_copy(cv, tbl2.at[pl.ds(r0, npt)])
    for k in range(npt // K):
        pltpu.sync_copy(zbuf, acc.at[pl.ds(r0 + k * K, K)])
    plsc.subcore_barrier()

    _edge_loop(nec, tbl2, sidx, didx, rows, acc, semg, sems_)
    plsc.subcore_barrier()
    pltpu.sync_copy(acc.at[pl.ds(r0, npt)], agg2_out.at[pl.ds(r0, npt)])


@jax.jit
def kernel(structure, H, input_weight, W1, b1, W2, b2):
    N, D = H.shape
    HID = W1.shape[1]
    C = W2.shape[1]
    E = structure.shape[1]

    NP = ((N + 1 + NS * K - 1) // (NS * K)) * (NS * K)
    EPW = -(-E // (NW * K * NBUF)) * (K * NBUF)
    NEC = EPW // K
    NPT = NP // NS

    wv = jnp.pad(input_weight, (0, NP - N))[:, None]
    edges = jnp.pad(structure, ((0, 0), (0, EPW * NW - E)),
                    constant_values=N)
    srcr = edges[0].reshape(NC, NS, NEC, K)
    dstr = edges[1].reshape(NC, NS, NEC, K)

    B = 2048
    grid = NP // B

    x1b, xs, wm = pl.pallas_call(
        _mm1_body,
        grid=(grid,),
        in_specs=[
            pl.BlockSpec((B, D), lambda i: (i, 0)),
            pl.BlockSpec((D, HID), lambda i: (0, 0)),
            pl.BlockSpec((B, 1), lambda i: (i, 0)),
            pl.BlockSpec((1, HID), lambda i: (0, 0)),
        ],
        out_specs=[
            pl.BlockSpec((B, HID), lambda i: (i, 0)),
            pl.BlockSpec((B, HID), lambda i: (i, 0)),
            pl.BlockSpec((B, HID), lambda i: (i, 0)),
        ],
        out_shape=[
            jax.ShapeDtypeStruct((NP, HID), jnp.float32),
            jax.ShapeDtypeStruct((NP, HID), jnp.float32),
            jax.ShapeDtypeStruct((NP, HID), jnp.float32),
        ],
    )(H, W1, wv, b1[None, :])

    h1, agg2 = pl.kernel(
        functools.partial(_mega_body, NEC, NPT, HID),
        out_type=[
            jax.ShapeDtypeStruct((NP, HID), jnp.float32),
            jax.ShapeDtypeStruct((NP, HID), jnp.float32),
        ],
        mesh=plsc.VectorSubcoreMesh(
            core_axis_name="c", subcore_axis_name="s",
            num_cores=NC, num_subcores=NS),
        scratch_types=[
            pltpu.VMEM((NEC, K), jnp.int32),
            pltpu.VMEM((NEC, K), jnp.int32),
            pltpu.VMEM((NBUF, K, HID), jnp.float32),
            pltpu.VMEM((K, HID), jnp.float32),
            pltpu.VMEM((NPT, HID), jnp.float32),
            pltpu.VMEM((NPT, HID), jnp.float32),
            pltpu.VMEM((NPT, HID), jnp.float32),
            pltpu.VMEM_SHARED((NP, HID), jnp.float32),
            pltpu.VMEM_SHARED((NP, HID), jnp.float32),
            pltpu.VMEM_SHARED((NP, HID), jnp.float32),
        ] + [pltpu.SemaphoreType.DMA] * (2 * NBUF),
        compiler_params=pltpu.CompilerParams(use_tc_tiling_on_sc=False),
    )(xs, x1b, wm, srcr, dstr)

    out = pl.pallas_call(
        _out_body,
        grid=(grid,),
        in_specs=[
            pl.BlockSpec((B, HID), lambda i: (i, 0)),
            pl.BlockSpec((B, HID), lambda i: (i, 0)),
            pl.BlockSpec((HID, C), lambda i: (0, 0)),
            pl.BlockSpec((1, C), lambda i: (0, 0)),
        ],
        out_specs=pl.BlockSpec((B, C), lambda i: (i, 0)),
        out_shape=jax.ShapeDtypeStruct((N, C), jnp.float32),
    )(agg2, h1, W2, b2[None, :])

    return out


# speedup vs baseline: 1.4164x; 1.0010x over previous
import functools

import jax
import jax.numpy as jnp
from jax import lax
from jax.experimental import pallas as pl
from jax.experimental.pallas import tpu as pltpu
from jax.experimental.pallas import tpu_sc as plsc

NC = 1
NS = 16
NW = NC * NS
K = 128
NBUF = 8


def _mm1_body(h_ref, w1_ref, wv_ref, b1_ref, x1b_ref, xs_ref, wm_ref):
    x1 = jnp.dot(h_ref[...], w1_ref[...], preferred_element_type=jnp.float32)
    wv = wv_ref[...]
    x1b_ref[...] = x1 + b1_ref[...]
    xs_ref[...] = x1 * wv
    wm_ref[...] = jnp.broadcast_to(wv, x1.shape)


def _out_body(agg_ref, h1_ref, w2_ref, b2_ref, o_ref):
    z = agg_ref[...] + h1_ref[...]
    z = jnp.dot(z, w2_ref[...], preferred_element_type=jnp.float32) + b2_ref[...]
    z = z - jnp.max(z, axis=1, keepdims=True)
    o_ref[...] = z - jnp.log(jnp.sum(jnp.exp(z), axis=1, keepdims=True))


def _edge_loop(nec, table, sidx, didx, rows, acc, semg, sems_):
    for b in range(NBUF):
        pltpu.async_copy(table.at[sidx.at[b]], rows.at[b], semg[b])

    def outer(o, carry):
        base = o * NBUF
        for b in range(NBUF):
            j = base + b
            pltpu.make_async_copy(table.at[sidx.at[j]], rows.at[b],
                                  semg[b]).wait()
            pltpu.async_copy(rows.at[b], acc.at[didx.at[j]], sems_[b],
                             add=True)
        for b in range(NBUF):
            j = base + b
            pltpu.make_async_copy(rows.at[b], acc.at[didx.at[j]],
                                  sems_[b]).wait()
            jn = j + NBUF

            @pl.when(jn < nec)
            def _():
                pltpu.async_copy(table.at[sidx.at[jn]], rows.at[b], semg[b])
        return carry

    lax.fori_loop(0, nec // NBUF, outer, 0)


def _mega_body(nec, npt, hid, xs, x1b, wm, srcr, dstr, h1_out, agg2_out,
               sidx, didx, rows, zbuf, av, bv, cv, acc, tbl1, tbl2, *sems):
    semg, sems_ = sems[:NBUF], sems[NBUF:]
    sid = lax.axis_index("s")
    r0 = sid * npt
    cps = [
        pltpu.async_copy(srcr.at[0, sid], sidx, semg[0]),
        pltpu.async_copy(dstr.at[0, sid], didx, semg[1]),
        pltpu.async_copy(xs.at[pl.ds(r0, npt)], tbl1.at[pl.ds(r0, npt)],
                         semg[2]),
    ]
    zero = jnp.zeros((hid,), jnp.float32)
    for i in range(K):
        zbuf[i, :] = zero
    for k in range(npt // K):
        pltpu.sync_copy(zbuf, acc.at[pl.ds(r0 + k * K, K)])
    for cp in cps:
        cp.wait()
    plsc.subcore_barrier()

    _edge_loop(nec, tbl1, sidx, didx, rows, acc, semg, sems_)
    plsc.subcore_barrier()

    nh = npt // 2

    def mrow(i, carry):
        h = jnp.maximum(av[i, :] + bv[i, :], 0.0)
        bv[i, :] = h
        cv[i, :] = h * cv[i, :]
        return carry

    for half in range(2):
        rh = r0 + half * nh
        pltpu.sync_copy(acc.at[pl.ds(rh, nh)], av)
        pltpu.sync_copy(x1b.at[pl.ds(rh, nh)], bv)
        pltpu.sync_copy(wm.at[pl.ds(rh, nh)], cv)
        lax.fori_loop(0, nh, mrow, 0)
        pltpu.sync_copy(bv, h1_out.at[pl.ds(rh, nh)])
        pltpu.sync_copy(cv, tbl2.at[pl.ds(rh, nh)])
    for k in range(npt // K):
        pltpu.sync_copy(zbuf, acc.at[pl.ds(r0 + k * K, K)])
    plsc.subcore_barrier()

    _edge_loop(nec, tbl2, sidx, didx, rows, acc, semg, sems_)
    plsc.subcore_barrier()
    pltpu.sync_copy(acc.at[pl.ds(r0, npt)], agg2_out.at[pl.ds(r0, npt)])


@jax.jit
def kernel(structure, H, input_weight, W1, b1, W2, b2):
    N, D = H.shape
    HID = W1.shape[1]
    C = W2.shape[1]
    E = structure.shape[1]

    NP = ((N + 1 + NS * K - 1) // (NS * K)) * (NS * K)
    EPW = -(-E // (NW * K * NBUF)) * (K * NBUF)
    NEC = EPW // K
    NPT = NP // NS

    wv = jnp.pad(input_weight, (0, NP - N))[:, None]
    edges = jnp.pad(structure, ((0, 0), (0, EPW * NW - E)),
                    constant_values=N)
    srcr = edges[0].reshape(NC, NS, NEC, K)
    dstr = edges[1].reshape(NC, NS, NEC, K)

    B = 2048
    grid = NP // B

    x1b, xs, wm = pl.pallas_call(
        _mm1_body,
        grid=(grid,),
        in_specs=[
            pl.BlockSpec((B, D), lambda i: (i, 0)),
            pl.BlockSpec((D, HID), lambda i: (0, 0)),
            pl.BlockSpec((B, 1), lambda i: (i, 0)),
            pl.BlockSpec((1, HID), lambda i: (0, 0)),
        ],
        out_specs=[
            pl.BlockSpec((B, HID), lambda i: (i, 0)),
            pl.BlockSpec((B, HID), lambda i: (i, 0)),
            pl.BlockSpec((B, HID), lambda i: (i, 0)),
        ],
        out_shape=[
            jax.ShapeDtypeStruct((NP, HID), jnp.float32),
            jax.ShapeDtypeStruct((NP, HID), jnp.float32),
            jax.ShapeDtypeStruct((NP, HID), jnp.float32),
        ],
    )(H, W1, wv, b1[None, :])

    h1, agg2 = pl.kernel(
        functools.partial(_mega_body, NEC, NPT, HID),
        out_type=[
            jax.ShapeDtypeStruct((NP, HID), jnp.float32),
            jax.ShapeDtypeStruct((NP, HID), jnp.float32),
        ],
        mesh=plsc.VectorSubcoreMesh(
            core_axis_name="c", subcore_axis_name="s",
            num_cores=NC, num_subcores=NS),
        scratch_types=[
            pltpu.VMEM((NEC, K), jnp.int32),
            pltpu.VMEM((NEC, K), jnp.int32),
            pltpu.VMEM((NBUF, K, HID), jnp.float32),
            pltpu.VMEM((K, HID), jnp.float32),
            pltpu.VMEM((NPT // 2, HID), jnp.float32),
            pltpu.VMEM((NPT // 2, HID), jnp.float32),
            pltpu.VMEM((NPT // 2, HID), jnp.float32),
            pltpu.VMEM_SHARED((NP, HID), jnp.float32),
            pltpu.VMEM_SHARED((NP, HID), jnp.float32),
            pltpu.VMEM_SHARED((NP, HID), jnp.float32),
        ] + [pltpu.SemaphoreType.DMA] * (2 * NBUF),
        compiler_params=pltpu.CompilerParams(use_tc_tiling_on_sc=False),
    )(xs, x1b, wm, srcr, dstr)

    out = pl.pallas_call(
        _out_body,
        grid=(grid,),
        in_specs=[
            pl.BlockSpec((B, HID), lambda i: (i, 0)),
            pl.BlockSpec((B, HID), lambda i: (i, 0)),
            pl.BlockSpec((HID, C), lambda i: (0, 0)),
            pl.BlockSpec((1, C), lambda i: (0, 0)),
        ],
        out_specs=pl.BlockSpec((B, C), lambda i: (i, 0)),
        out_shape=jax.ShapeDtypeStruct((N, C), jnp.float32),
    )(agg2, h1, W2, b2[None, :])

    return out


# speedup vs baseline: 1.4409x; 1.0173x over previous
import functools

import jax
import jax.numpy as jnp
from jax import lax
from jax.experimental import pallas as pl
from jax.experimental.pallas import tpu as pltpu
from jax.experimental.pallas import tpu_sc as plsc

NC = 1
NS = 16
NW = NC * NS
K = 128
NBUF = 8


def _mm1_body(h_ref, w1_ref, wv_ref, b1_ref, x1b_ref, xs_ref, wm_ref):
    x1 = jnp.dot(h_ref[...], w1_ref[...], preferred_element_type=jnp.float32)
    wv = wv_ref[...]
    x1b_ref[...] = x1 + b1_ref[...]
    xs_ref[...] = x1 * wv
    wm_ref[...] = jnp.broadcast_to(wv, x1.shape)


def _out_body(agg_ref, h1_ref, w2_ref, b2_ref, o_ref):
    z = agg_ref[...] + h1_ref[...]
    z = jnp.dot(z, w2_ref[...], preferred_element_type=jnp.float32) + b2_ref[...]
    z = z - jnp.max(z, axis=1, keepdims=True)
    o_ref[...] = z - jnp.log(jnp.sum(jnp.exp(z), axis=1, keepdims=True))


def _edge_loop(nec, table, sidx, didx, rows, acc, semg, sems_):
    for b in range(NBUF):
        pltpu.async_copy(table.at[sidx.at[b]], rows.at[b], semg[b])

    def outer(o, carry):
        base = o * NBUF
        for b in range(NBUF):
            j = base + b
            pltpu.make_async_copy(table.at[sidx.at[j]], rows.at[b],
                                  semg[b]).wait()
            pltpu.async_copy(rows.at[b], acc.at[didx.at[j]], sems_[b],
                             add=True)
        for b in range(NBUF):
            j = base + b
            pltpu.make_async_copy(rows.at[b], acc.at[didx.at[j]],
                                  sems_[b]).wait()
            jn = j + NBUF

            @pl.when(jn < nec)
            def _():
                pltpu.async_copy(table.at[sidx.at[jn]], rows.at[b], semg[b])
        return carry

    lax.fori_loop(0, nec // NBUF, outer, 0)


def _mega_body(nec, npt, hid, xs, x1b, wm, srcr, dstr, h1_out, agg2_out,
               sidx, didx, rows, zbuf, av, bv, cv, acc, tbl1, tbl2, *sems):
    semg, sems_ = sems[:NBUF], sems[NBUF:]
    sid = lax.axis_index("s")
    r0 = sid * npt
    cps = [
        pltpu.async_copy(srcr.at[0, sid], sidx, semg[0]),
        pltpu.async_copy(dstr.at[0, sid], didx, semg[1]),
        pltpu.async_copy(xs.at[pl.ds(r0, npt)], tbl1.at[pl.ds(r0, npt)],
                         semg[2]),
    ]
    zero = jnp.zeros((hid,), jnp.float32)
    for i in range(K):
        zbuf[i, :] = zero
    for k in range(npt // K):
        pltpu.sync_copy(zbuf, acc.at[pl.ds(r0 + k * K, K)])
    for cp in cps:
        cp.wait()
    plsc.subcore_barrier()

    _edge_loop(nec, tbl1, sidx, didx, rows, acc, semg, sems_)
    plsc.subcore_barrier()

    nh = npt // 2

    def mrow(i, carry):
        for u in range(4):
            r = i * 4 + u
            h = jnp.maximum(av[r, :] + bv[r, :], 0.0)
            bv[r, :] = h
            cv[r, :] = h * cv[r, :]
        return carry

    for half in range(2):
        rh = r0 + half * nh
        pltpu.sync_copy(acc.at[pl.ds(rh, nh)], av)
        pltpu.sync_copy(x1b.at[pl.ds(rh, nh)], bv)
        pltpu.sync_copy(wm.at[pl.ds(rh, nh)], cv)
        lax.fori_loop(0, nh // 4, mrow, 0)
        pltpu.sync_copy(bv, h1_out.at[pl.ds(rh, nh)])
        pltpu.sync_copy(cv, tbl2.at[pl.ds(rh, nh)])
    for k in range(npt // K):
        pltpu.sync_copy(zbuf, acc.at[pl.ds(r0 + k * K, K)])
    plsc.subcore_barrier()

    _edge_loop(nec, tbl2, sidx, didx, rows, acc, semg, sems_)
    plsc.subcore_barrier()
    pltpu.sync_copy(acc.at[pl.ds(r0, npt)], agg2_out.at[pl.ds(r0, npt)])


@jax.jit
def kernel(structure, H, input_weight, W1, b1, W2, b2):
    N, D = H.shape
    HID = W1.shape[1]
    C = W2.shape[1]
    E = structure.shape[1]

    NP = ((N + 1 + NS * K - 1) // (NS * K)) * (NS * K)
    EPW = -(-E // (NW * K * NBUF)) * (K * NBUF)
    NEC = EPW // K
    NPT = NP // NS

    wv = jnp.pad(input_weight, (0, NP - N))[:, None]
    edges = jnp.pad(structure, ((0, 0), (0, EPW * NW - E)),
                    constant_values=N)
    srcr = edges[0].reshape(NC, NS, NEC, K)
    dstr = edges[1].reshape(NC, NS, NEC, K)

    B = 2048
    grid = NP // B

    x1b, xs, wm = pl.pallas_call(
        _mm1_body,
        grid=(grid,),
        in_specs=[
            pl.BlockSpec((B, D), lambda i: (i, 0)),
            pl.BlockSpec((D, HID), lambda i: (0, 0)),
            pl.BlockSpec((B, 1), lambda i: (i, 0)),
            pl.BlockSpec((1, HID), lambda i: (0, 0)),
        ],
        out_specs=[
            pl.BlockSpec((B, HID), lambda i: (i, 0)),
            pl.BlockSpec((B, HID), lambda i: (i, 0)),
            pl.BlockSpec((B, HID), lambda i: (i, 0)),
        ],
        out_shape=[
            jax.ShapeDtypeStruct((NP, HID), jnp.float32),
            jax.ShapeDtypeStruct((NP, HID), jnp.float32),
            jax.ShapeDtypeStruct((NP, HID), jnp.float32),
        ],
    )(H, W1, wv, b1[None, :])

    h1, agg2 = pl.kernel(
        functools.partial(_mega_body, NEC, NPT, HID),
        out_type=[
            jax.ShapeDtypeStruct((NP, HID), jnp.float32),
            jax.ShapeDtypeStruct((NP, HID), jnp.float32),
        ],
        mesh=plsc.VectorSubcoreMesh(
            core_axis_name="c", subcore_axis_name="s",
            num_cores=NC, num_subcores=NS),
        scratch_types=[
            pltpu.VMEM((NEC, K), jnp.int32),
            pltpu.VMEM((NEC, K), jnp.int32),
            pltpu.VMEM((NBUF, K, HID), jnp.float32),
            pltpu.VMEM((K, HID), jnp.float32),
            pltpu.VMEM((NPT // 2, HID), jnp.float32),
            pltpu.VMEM((NPT // 2, HID), jnp.float32),
            pltpu.VMEM((NPT // 2, HID), jnp.float32),
            pltpu.VMEM_SHARED((NP, HID), jnp.float32),
            pltpu.VMEM_SHARED((NP, HID), jnp.float32),
            pltpu.VMEM_SHARED((NP, HID), jnp.float32),
        ] + [pltpu.SemaphoreType.DMA] * (2 * NBUF),
        compiler_params=pltpu.CompilerParams(use_tc_tiling_on_sc=False),
    )(xs, x1b, wm, srcr, dstr)

    out = pl.pallas_call(
        _out_body,
        grid=(grid,),
        in_specs=[
            pl.BlockSpec((B, HID), lambda i: (i, 0)),
            pl.BlockSpec((B, HID), lambda i: (i, 0)),
            pl.BlockSpec((HID, C), lambda i: (0, 0)),
            pl.BlockSpec((1, C), lambda i: (0, 0)),
        ],
        out_specs=pl.BlockSpec((B, C), lambda i: (i, 0)),
        out_shape=jax.ShapeDtypeStruct((N, C), jnp.float32),
    )(agg2, h1, W2, b2[None, :])

    return out
